# Initial kernel scaffold; baseline (speedup 1.0000x reference)
import jax
import jax.numpy as jnp
from jax.experimental import pallas as pl


def kernel(x, edge_index, batch_idx, gW1_0, gb1_0, gW2_0, gb2_0, gW1_1, gb1_1, gW2_1, gb2_1, gW1_2, gb1_2, gW2_2, gb2_2, cW1, cb1, cW2, cb2, cW3, cb3, aW, ab, eW, eb):
    raise NotImplementedError("write your pallas kernel here")


# speedup vs baseline: 3.7786x; 3.7786x over previous
import functools

import jax
import jax.numpy as jnp
from jax import lax
from jax.experimental import pallas as pl
from jax.experimental.pallas import tpu as pltpu
from jax.experimental.pallas import tpu_sc as plsc

N = 10000
E = 320000
G = 8
CHUNK = 128
NSUB = 16
CPT = 160
EPAD = NSUB * CPT * CHUNK
ROWS_PT = 624
ROWS_TAIL = N - NSUB * ROWS_PT
NACC = N + 8
KB = 16


def _edge_loop(table, acc, src3, dst3, s, j0, n_macro,
               sidx, didx, rbuf0, rbuf1, gsem0, gsem1, ssem0, ssem1):
    def macro(k, carry):
        base = j0 + k * KB
        base = pl.multiple_of(base, 8)
        pltpu.sync_copy(src3.at[s, pl.ds(base, KB)], sidx)
        pltpu.sync_copy(dst3.at[s, pl.ds(base, KB)], didx)
        g = {0: pltpu.async_copy(table.at[sidx.at[0]], rbuf0, gsem0),
             1: pltpu.async_copy(table.at[sidx.at[1]], rbuf1, gsem1)}
        for j in range(0, KB, 2):
            g[0].wait()
            pltpu.sync_copy(rbuf0, acc.at[didx.at[j]], add=True)
            if j + 2 < KB:
                g[0] = pltpu.async_copy(table.at[sidx.at[j + 2]], rbuf0, gsem0)
            g[1].wait()
            pltpu.sync_copy(rbuf1, acc.at[didx.at[j + 1]], add=True)
            if j + 3 < KB:
                g[1] = pltpu.async_copy(table.at[sidx.at[j + 3]], rbuf1, gsem1)
        return carry

    lax.fori_loop(0, n_macro, macro, 0)


def _make_sc_agg(d2):
    mesh = plsc.VectorSubcoreMesh(core_axis_name="c", subcore_axis_name="s",
                                  num_cores=2, num_subcores=NSUB)

    @functools.partial(
        pl.kernel,
        out_type=[jax.ShapeDtypeStruct((N, d2), jnp.float32),
                  jax.ShapeDtypeStruct((N, d2), jnp.float32)],
        mesh=mesh,
        scratch_types=[
            pltpu.VMEM((KB, CHUNK), jnp.int32),
            pltpu.VMEM((KB, CHUNK), jnp.int32),
            pltpu.VMEM((CHUNK, d2), jnp.float32),
            pltpu.VMEM((CHUNK, d2), jnp.float32),
            pltpu.VMEM_SHARED((NACC, d2), jnp.float32),
            pltpu.SemaphoreType.DMA,
            pltpu.SemaphoreType.DMA,
            pltpu.SemaphoreType.DMA,
            pltpu.SemaphoreType.DMA,
        ],
    )
    def sc_agg(hL, hR, src3, dst3, outL, outR,
               sidx, didx, rbuf0, rbuf1, acc, gsem0, gsem1, ssem0, ssem1):
        s = lax.axis_index("s")
        c = lax.axis_index("c")
        row0 = s * ROWS_PT
        row0 = pl.multiple_of(row0, 8)

        def run(table, out):
            pltpu.sync_copy(table.at[pl.ds(row0, ROWS_PT)],
                            acc.at[pl.ds(row0, ROWS_PT)])

            @pl.when(s == 0)
            def _():
                pltpu.sync_copy(table.at[pl.ds(NSUB * ROWS_PT, ROWS_TAIL)],
                                acc.at[pl.ds(NSUB * ROWS_PT, ROWS_TAIL)])

            plsc.subcore_barrier()
            _edge_loop(table, acc, src3, dst3, s, 0, CPT // KB,
                       sidx, didx, rbuf0, rbuf1, gsem0, gsem1, ssem0, ssem1)
            plsc.subcore_barrier()
            pltpu.sync_copy(acc.at[pl.ds(row0, ROWS_PT)],
                            out.at[pl.ds(row0, ROWS_PT)])

            @pl.when(s == 0)
            def _():
                pltpu.sync_copy(acc.at[pl.ds(NSUB * ROWS_PT, ROWS_TAIL)],
                                out.at[pl.ds(NSUB * ROWS_PT, ROWS_TAIL)])

        @pl.when(c == 0)
        def _():
            run(hL, outL)

        @pl.when(c == 1)
        def _():
            run(hR, outR)

    return sc_agg


def _make_sc_agg_es():
    mesh = plsc.VectorSubcoreMesh(core_axis_name="c", subcore_axis_name="s",
                                  num_cores=2, num_subcores=NSUB)
    half = CPT // 2

    @functools.partial(
        pl.kernel,
        out_type=[jax.ShapeDtypeStruct((N, 128), jnp.float32),
                  jax.ShapeDtypeStruct((N, 128), jnp.float32)],
        mesh=mesh,
        scratch_types=[
            pltpu.VMEM((KB, CHUNK), jnp.int32),
            pltpu.VMEM((KB, CHUNK), jnp.int32),
            pltpu.VMEM((CHUNK, 128), jnp.float32),
            pltpu.VMEM((CHUNK, 128), jnp.float32),
            pltpu.VMEM_SHARED((NACC, 128), jnp.float32),
            pltpu.SemaphoreType.DMA,
            pltpu.SemaphoreType.DMA,
            pltpu.SemaphoreType.DMA,
            pltpu.SemaphoreType.DMA,
        ],
    )
    def sc_agg_es(h, zeros, src3, dst3, out0, out1,
                  sidx, didx, rbuf0, rbuf1, acc, gsem0, gsem1, ssem0, ssem1):
        s = lax.axis_index("s")
        c = lax.axis_index("c")
        row0 = s * ROWS_PT
        row0 = pl.multiple_of(row0, 8)

        def run(init_src, out, j0):
            pltpu.sync_copy(init_src.at[pl.ds(row0, ROWS_PT)],
                            acc.at[pl.ds(row0, ROWS_PT)])

            @pl.when(s == 0)
            def _():
                pltpu.sync_copy(init_src.at[pl.ds(NSUB * ROWS_PT, ROWS_TAIL)],
                                acc.at[pl.ds(NSUB * ROWS_PT, ROWS_TAIL)])

            plsc.subcore_barrier()
            _edge_loop(h, acc, src3, dst3, s, j0, half // KB,
                       sidx, didx, rbuf0, rbuf1, gsem0, gsem1, ssem0, ssem1)
            plsc.subcore_barrier()
            pltpu.sync_copy(acc.at[pl.ds(row0, ROWS_PT)],
                            out.at[pl.ds(row0, ROWS_PT)])

            @pl.when(s == 0)
            def _():
                pltpu.sync_copy(acc.at[pl.ds(NSUB * ROWS_PT, ROWS_TAIL)],
                                out.at[pl.ds(NSUB * ROWS_PT, ROWS_TAIL)])

        @pl.when(c == 0)
        def _():
            run(h, out0, 0)

        @pl.when(c == 1)
        def _():
            run(zeros, out1, half)

    return sc_agg_es


def _make_mlp0():
    R = 1000

    def body(a0_ref, a1_ref, W1_ref, b1_ref, W2_ref, b2_ref, outL_ref, outR_ref):
        m = a0_ref[...] + a1_ref[...]
        z = jnp.dot(m, W1_ref[...], preferred_element_type=jnp.float32)
        z = jnp.maximum(z + b1_ref[...], 0.0)
        y = jnp.dot(z, W2_ref[...], preferred_element_type=jnp.float32)
        y = jnp.maximum(y + b2_ref[...], 0.0)
        outL_ref[...] = y[:, :128]
        outR_ref[...] = y[:, 128:]

    return pl.pallas_call(
        body,
        grid=(N // R,),
        in_specs=[
            pl.BlockSpec((R, 128), lambda i: (i, 0)),
            pl.BlockSpec((R, 128), lambda i: (i, 0)),
            pl.BlockSpec((128, 256), lambda i: (0, 0)),
            pl.BlockSpec((1, 256), lambda i: (0, 0)),
            pl.BlockSpec((256, 256), lambda i: (0, 0)),
            pl.BlockSpec((1, 256), lambda i: (0, 0)),
        ],
        out_specs=[pl.BlockSpec((R, 128), lambda i: (i, 0)),
                   pl.BlockSpec((R, 128), lambda i: (i, 0))],
        out_shape=[jax.ShapeDtypeStruct((N, 128), jnp.float32)] * 2,
    )


def _make_mlp(d2in):
    R = 1000
    d_in = 2 * d2in

    def body(mL_ref, mR_ref, W1_ref, b1_ref, W2_ref, b2_ref, outL_ref, outR_ref):
        z = jnp.dot(mL_ref[...], W1_ref[:d2in, :],
                    preferred_element_type=jnp.float32)
        z = z + jnp.dot(mR_ref[...], W1_ref[d2in:, :],
                        preferred_element_type=jnp.float32)
        z = jnp.maximum(z + b1_ref[...], 0.0)
        y = jnp.dot(z, W2_ref[...], preferred_element_type=jnp.float32)
        y = jnp.maximum(y + b2_ref[...], 0.0)
        outL_ref[...] = y[:, :128]
        outR_ref[...] = y[:, 128:]

    return pl.pallas_call(
        body,
        grid=(N // R,),
        in_specs=[
            pl.BlockSpec((R, d2in), lambda i: (i, 0)),
            pl.BlockSpec((R, d2in), lambda i: (i, 0)),
            pl.BlockSpec((d_in, 256), lambda i: (0, 0)),
            pl.BlockSpec((1, 256), lambda i: (0, 0)),
            pl.BlockSpec((256, 256), lambda i: (0, 0)),
            pl.BlockSpec((1, 256), lambda i: (0, 0)),
        ],
        out_specs=[pl.BlockSpec((R, 128), lambda i: (i, 0)),
                   pl.BlockSpec((R, 128), lambda i: (i, 0))],
        out_shape=[jax.ShapeDtypeStruct((N, 128), jnp.float32)] * 2,
    )


def _head_body(hL_ref, hR_ref, bf_ref,
               cW1_ref, cb1_ref, cW2_ref, cb2_ref, cW3_ref, cb3_ref,
               aW_ref, ab_ref, eW_ref, eb_ref,
               logits_ref, probs_ref, api_ref, ent_ref,
               sumsL, sumsR, cnt, nsumL, nsumR):
    i = pl.program_id(0)
    R = 1000

    @pl.when(i == 0)
    def _():
        sumsL[...] = jnp.zeros_like(sumsL)
        sumsR[...] = jnp.zeros_like(sumsR)
        cnt[...] = jnp.zeros_like(cnt)
        nsumL[...] = jnp.zeros_like(nsumL)
        nsumR[...] = jnp.zeros_like(nsumR)

    hL = hL_ref[...]
    hR = hR_ref[...]
    bf = bf_ref[...]
    gid = lax.broadcasted_iota(jnp.int32, (1, G), 1).astype(jnp.float32)
    oh = (bf == gid).astype(jnp.float32)
    dn = (((0,), (0,)), ((), ()))
    sumsL[...] += lax.dot_general(oh, hL, dn, preferred_element_type=jnp.float32)
    sumsR[...] += lax.dot_general(oh, hR, dn, preferred_element_type=jnp.float32)
    cnt[...] += lax.dot_general(oh, jnp.ones((R, 1), jnp.float32), dn,
                                preferred_element_type=jnp.float32)
    nsumL[...] += jnp.sum(hL, axis=0, keepdims=True)
    nsumR[...] += jnp.sum(hR, axis=0, keepdims=True)

    @pl.when(i == (N // R) - 1)
    def _():
        inv = 1.0 / jnp.maximum(cnt[...], 1.0)
        geL = sumsL[...] * inv
        geR = sumsR[...] * inv
        z1 = jnp.dot(geL, cW1_ref[:128, :], preferred_element_type=jnp.float32)
        z1 = z1 + jnp.dot(geR, cW1_ref[128:, :], preferred_element_type=jnp.float32)
        z1 = jnp.maximum(z1 + cb1_ref[...], 0.0)
        z2 = jnp.dot(z1, cW2_ref[...], preferred_element_type=jnp.float32)
        z2 = jnp.maximum(z2 + cb2_ref[...], 0.0)
        logits = jnp.dot(z2, cW3_ref[...], preferred_element_type=jnp.float32)
        logits = logits + cb3_ref[...]
        m = jnp.max(logits, axis=-1, keepdims=True)
        ex = jnp.exp(logits - m)
        probs = ex / jnp.sum(ex, axis=-1, keepdims=True)
        nmL = nsumL[...] * (1.0 / N)
        nmR = nsumR[...] * (1.0 / N)
        api = jnp.dot(nmL, aW_ref[:128, :], preferred_element_type=jnp.float32)
        api = api + jnp.dot(nmR, aW_ref[128:, :], preferred_element_type=jnp.float32)
        api = api + ab_ref[...]
        ent = jnp.dot(nmL, eW_ref[:128, :], preferred_element_type=jnp.float32)
        ent = ent + jnp.dot(nmR, eW_ref[128:, :], preferred_element_type=jnp.float32)
        ent = ent + eb_ref[...]
        logits_ref[...] = logits
        probs_ref[...] = probs
        api_ref[...] = api
        ent_ref[...] = ent


def _make_head():
    R = 1000
    full = lambda shape: pl.BlockSpec(shape, lambda i: tuple(0 for _ in shape))
    return pl.pallas_call(
        _head_body,
        grid=(N // R,),
        in_specs=[
            pl.BlockSpec((R, 128), lambda i: (i, 0)),
            pl.BlockSpec((R, 128), lambda i: (i, 0)),
            pl.BlockSpec((R, 1), lambda i: (i, 0)),
            full((256, 256)), full((1, 256)),
            full((256, 128)), full((1, 128)),
            full((128, 2)), full((1, 2)),
            full((256, 20)), full((1, 20)),
            full((256, 1)), full((1, 1)),
        ],
        out_specs=[full((G, 2)), full((G, 2)), full((1, 20)), full((1, 1))],
        out_shape=[jax.ShapeDtypeStruct((G, 2), jnp.float32),
                   jax.ShapeDtypeStruct((G, 2), jnp.float32),
                   jax.ShapeDtypeStruct((1, 20), jnp.float32),
                   jax.ShapeDtypeStruct((1, 1), jnp.float32)],
        scratch_shapes=[
            pltpu.VMEM((G, 128), jnp.float32),
            pltpu.VMEM((G, 128), jnp.float32),
            pltpu.VMEM((G, 1), jnp.float32),
            pltpu.VMEM((1, 128), jnp.float32),
            pltpu.VMEM((1, 128), jnp.float32),
        ],
    )


_sc_cache = {}


def _sc_agg(d2):
    if d2 not in _sc_cache:
        _sc_cache[d2] = _make_sc_agg(d2)
    return _sc_cache[d2]


def _sc_agg_es():
    if "es" not in _sc_cache:
        _sc_cache["es"] = _make_sc_agg_es()
    return _sc_cache["es"]


_mlp0 = _make_mlp0()
_mlp128 = _make_mlp(128)
_head = _make_head()


def kernel(x, edge_index, batch_idx,
           gW1_0, gb1_0, gW2_0, gb2_0,
           gW1_1, gb1_1, gW2_1, gb2_1,
           gW1_2, gb1_2, gW2_2, gb2_2,
           cW1, cb1, cW2, cb2, cW3, cb3,
           aW, ab, eW, eb):
    pad = EPAD - E
    srcp = jnp.concatenate([edge_index[0], jnp.zeros((pad,), jnp.int32)])
    dstp = jnp.concatenate([edge_index[1], jnp.full((pad,), N, jnp.int32)])
    src3 = srcp.reshape(CPT, NSUB, CHUNK).transpose(1, 0, 2)
    dst3 = dstp.reshape(CPT, NSUB, CHUNK).transpose(1, 0, 2)

    zeros = jnp.zeros((N, 128), jnp.float32)
    a0, a1 = _sc_agg_es()(x, zeros, src3, dst3)
    h1L, h1R = _mlp0(a0, a1, gW1_0, gb1_0[None, :], gW2_0, gb2_0[None, :])
    m1L, m1R = _sc_agg(128)(h1L, h1R, src3, dst3)
    h2L, h2R = _mlp128(m1L, m1R, gW1_1, gb1_1[None, :], gW2_1, gb2_1[None, :])
    m2L, m2R = _sc_agg(128)(h2L, h2R, src3, dst3)
    h3L, h3R = _mlp128(m2L, m2R, gW1_2, gb1_2[None, :], gW2_2, gb2_2[None, :])

    bf = batch_idx.astype(jnp.float32)[:, None]
    logits, probs, api, ent = _head(
        h3L, h3R, bf,
        cW1, cb1[None, :], cW2, cb2[None, :], cW3, cb3[None, :],
        aW, ab[None, :], eW, eb[None, :])
    return logits, probs, api.reshape(20), ent.reshape(1)
